# baseline (speedup 1.0000x reference)
import numpy as np

import jax
import jax.numpy as jnp
from jax import lax
from jax.experimental import pallas as pl
from jax.experimental.pallas import tpu as pltpu
from jax.experimental.pallas import tpu_sc as plsc

_VOCAB = 1000000
_D = 64
_B = 1024
_NSAMP = 4096
_NCORES = 2
_NSUB = 16
_L = 16
_NW = _NCORES * _NSUB
_BPW = _B // _NW
_CHUNK = 128
_NCHUNK = _NSAMP // _CHUNK
_NBUF = 4

def _rotl(x, r):
    return (x << np.uint32(r)) | (x >> np.uint32(32 - r))


def _threefry2x32(k0, k1, x0, x1):
    x0 = x0.astype(np.uint32).copy()
    x1 = x1.astype(np.uint32).copy()
    ks = [np.uint32(k0), np.uint32(k1),
          np.uint32(k0) ^ np.uint32(k1) ^ np.uint32(0x1BD11BDA)]
    rotations = [(13, 15, 26, 6), (17, 29, 16, 24)]
    x0 += ks[0]
    x1 += ks[1]
    for i in range(5):
        for r in rotations[i % 2]:
            x0 += x1
            x1 = _rotl(x1, r)
            x1 ^= x0
        x0 += ks[(i + 1) % 3]
        x1 += ks[(i + 2) % 3] + np.uint32(i + 1)
    return x0, x1


def _random_bits32(k0, k1, size):
    b0, b1 = _threefry2x32(k0, k1, np.zeros(size, np.uint32),
                           np.arange(size, dtype=np.uint32))
    return b0 ^ b1


_samples_cache = []

_PB = 8192
_QB = _PB // 4
_NPB = (_VOCAB + _PB - 1) // _PB
_VPAD = _NPB * _PB


def _perm_np(j):
    i = j // _PB
    p = j % _PB
    return 4 * (i * _QB + p % _QB) + p // _QB


def _perm_vec(j):
    i = j // _PB
    p = j - i * _PB
    return 4 * (i * _QB + p % _QB) + p // _QB


def _get_samples():
    if not _samples_cache:
        b0, b1 = _threefry2x32(0, 1234, np.zeros(2, np.uint32),
                               np.arange(2, dtype=np.uint32))
        size = _B * _NSAMP
        hi = _random_bits32(b0[0], b1[0], size)
        lo = _random_bits32(b0[1], b1[1], size)
        span = np.uint32(_VOCAB)
        mult = np.uint32((2 ** 16) % _VOCAB)
        mult = np.uint32((int(mult) * int(mult)) % (2 ** 32)) % span
        off = ((hi % span) * mult + (lo % span)) % span
        _samples_cache.append(_perm_np(off.astype(np.int64)).astype(np.int32))
    return jnp.asarray(_samples_cache[0])


def _prenorm_body(x_ref, o_ref):
    x = x_ref[...]
    s = jnp.sum(x * x, axis=1, keepdims=True)
    o_ref[...] = x * lax.rsqrt(jnp.maximum(s, 1e-12))


def _prenorm(x, block_rows):
    rows = x.shape[0]
    grid = rows // block_rows
    return pl.pallas_call(
        _prenorm_body,
        grid=(grid,),
        in_specs=[pl.BlockSpec((block_rows, _D), lambda i: (i, 0))],
        out_specs=pl.BlockSpec((block_rows, _D), lambda i: (i, 0)),
        out_shape=jax.ShapeDtypeStruct((rows, _D), jnp.float32),
    )(x)


def _prenorm_proj_body(xt_ref, o_ref):
    xt = xt_ref[...]
    s = jnp.sum(xt * xt, axis=0, keepdims=True)
    xn = xt * lax.rsqrt(jnp.maximum(s, 1e-12))
    lo = lax.bitcast_convert_type(xn[0:_D // 2, :].astype(jnp.bfloat16),
                                  jnp.uint16).astype(jnp.int32)
    hi = lax.bitcast_convert_type(xn[_D // 2:_D, :].astype(jnp.bfloat16),
                                  jnp.uint16).astype(jnp.int32)
    packed = (lo | (hi << 16)).T
    for q in range(4):
        o_ref[:, q * 32:(q + 1) * 32] = packed[q * _QB:(q + 1) * _QB, :]


def _prenorm_proj(x):
    return pl.pallas_call(
        _prenorm_proj_body,
        grid=(_NPB,),
        in_specs=[pl.BlockSpec((_D, _PB), lambda i: (0, i))],
        out_specs=pl.BlockSpec((_QB, 128), lambda i: (i, 0)),
        out_shape=jax.ShapeDtypeStruct((_VPAD // 4, 128), jnp.int32),
    )(x.T)


def _sc_body(projn, inpn, labels, samples, noise_out, dlab_out,
             uin, lab, labrows, idx, rows0, rows1, rows2, rows3,
             nstage, dstage, scr, scr2, scr3, scr4,
             sem_l, s0, s1, s2, s3):
    w = lax.axis_index("s") * _NCORES + lax.axis_index("c")
    b0 = w * _BPW
    scrs = (scr, scr2, scr3, scr4)
    bufs = (rows0, rows1, rows2, rows3)
    sems = (s0, s1, s2, s3)
    iota = lax.iota(jnp.int32, _L)

    pltpu.sync_copy(inpn.at[pl.ds(b0, _BPW), :], uin)
    pltpu.sync_copy(labels.at[pl.ds(b0, _BPW)], lab)
    for g in range(_BPW // _L):
        lv = lab[pl.ds(g * _L, _L)]
        lab[pl.ds(g * _L, _L)] = _perm_vec(lv)
    pltpu.async_copy(projn.at[lab], labrows, sem_l).wait()

    def transpose_sum(S):
        dvs = []
        for j in range(_L):
            colrot = (iota + j) & (_L - 1)
            dvs.append(plsc.load_gather(S, [iota, colrot]))
        while len(dvs) > 1:
            dvs = [dvs[t] + dvs[t + 1] for t in range(0, len(dvs), 2)]
        return dvs[0]

    def unpack4(ref, row):
        a = ref[row, pl.ds(0, _L)]
        b = ref[row, pl.ds(_L, _L)]
        return (lax.bitcast_convert_type(lax.shift_left(a, 16), jnp.float32),
                lax.bitcast_convert_type(lax.shift_left(b, 16), jnp.float32),
                lax.bitcast_convert_type(a, jnp.float32),
                lax.bitcast_convert_type(b, jnp.float32))

    for g in range(_BPW // _L):
        for r in range(_L):
            row = g * _L + r
            vs = unpack4(labrows, row)
            pr = None
            for kd in range(_D // _L):
                uvec = uin[row, pl.ds(kd * _L, _L)]
                t = vs[kd] * uvec
                pr = t if pr is None else pr + t
            scr[r, :] = pr
        dstage[g, :] = transpose_sum(scr)

    def b_body(i, _):
        pltpu.sync_copy(samples.at[pl.ds((b0 + i) * _NSAMP, _NSAMP)], idx)
        us = [uin[i, pl.ds(k * _L, _L)] for k in range(_D // _L)]
        ua = plsc.pack(us[0], us[2], format=plsc.PackFormat.INTERLEAVED)
        ub = plsc.pack(us[1], us[3], format=plsc.PackFormat.INTERLEAVED)
        labv = plsc.load_gather(lab, [jnp.full((_L,), i, jnp.int32)])

        for k in range(_NBUF):
            pltpu.make_async_copy(projn.at[idx.at[pl.ds(k * _CHUNK, _CHUNK)]],
                                  bufs[k], sems[k]).start()

        def c_body(c0, acc):
            for k in range(_NBUF):
                c = c0 + k
                pltpu.make_async_copy(projn.at[idx.at[pl.ds(c * _CHUNK,
                                                            _CHUNK)]],
                                      bufs[k], sems[k]).wait()

                def g_body(m, acc2):
                    for half in range(4):
                        g = m * 4 + half
                        r0 = g * _L
                        sg = scrs[half]
                        for r8 in range(0, _L, 16):
                            avs = [plsc.bitcast(
                                bufs[k][r0 + r8 + t, pl.ds(0, _L)],
                                jnp.bfloat16) for t in range(16)]
                            bvs = [plsc.bitcast(
                                bufs[k][r0 + r8 + t, pl.ds(_L, _L)],
                                jnp.bfloat16) for t in range(16)]
                            ps = [avs[t] * ua for t in range(16)]
                            qs = [bvs[t] * ub for t in range(16)]
                            ss = [ps[t] + qs[t] for t in range(16)]
                            eo = [plsc.unpack(
                                ss[t], format=plsc.PackFormat.INTERLEAVED)
                                for t in range(16)]
                            prs = [eo[t][0] + eo[t][1] for t in range(16)]
                            for t in range(16):
                                sg[r8 + t, :] = prs[t]
                        dot = transpose_sum(sg)
                        ids16 = idx[pl.ds(c * _CHUNK + r0, _L)]
                        e = jnp.where(ids16 == labv, 0.0, jnp.exp(dot))
                        acc2 = acc2 + e
                    return acc2

                acc = lax.fori_loop(0, _CHUNK // (4 * _L), g_body, acc)

                nxt = c + _NBUF

                @pl.when(nxt < _NCHUNK)
                def _():
                    pltpu.make_async_copy(
                        projn.at[idx.at[pl.ds(nxt * _CHUNK, _CHUNK)]],
                        bufs[k], sems[k]).start()
            return acc

        acc = lax.fori_loop(0, _NCHUNK // _NBUF, c_body,
                            jnp.zeros((_L,), jnp.float32))
        nstage[i, :] = acc
        return 0

    lax.fori_loop(0, _BPW, b_body, 0)

    pltpu.sync_copy(nstage, noise_out.at[pl.ds(b0, _BPW), :])
    ng = _BPW // _L
    pltpu.sync_copy(dstage, dlab_out.at[pl.ds(w * ng, ng), :])


_sc_main = pl.kernel(
    _sc_body,
    out_type=(
        jax.ShapeDtypeStruct((_B, _L), jnp.float32),
        jax.ShapeDtypeStruct((_B // _L, _L), jnp.float32),
    ),
    mesh=plsc.VectorSubcoreMesh(core_axis_name="c", subcore_axis_name="s"),
    compiler_params=pltpu.CompilerParams(needs_layout_passes=False,
                                         use_tc_tiling_on_sc=False,
                                         disable_bounds_checks=True),
    scratch_types=[
        pltpu.VMEM((_BPW, _D), jnp.float32),
        pltpu.VMEM((_BPW,), jnp.int32),
        pltpu.VMEM((_BPW, _D // 2), jnp.int32),
        pltpu.VMEM((_NSAMP,), jnp.int32),
        pltpu.VMEM((_CHUNK, _D // 2), jnp.int32),
        pltpu.VMEM((_CHUNK, _D // 2), jnp.int32),
        pltpu.VMEM((_CHUNK, _D // 2), jnp.int32),
        pltpu.VMEM((_CHUNK, _D // 2), jnp.int32),
        pltpu.VMEM((_BPW, _L), jnp.float32),
        pltpu.VMEM((_BPW // _L, _L), jnp.float32),
        pltpu.VMEM((_L, _L), jnp.float32),
        pltpu.VMEM((_L, _L), jnp.float32),
        pltpu.VMEM((_L, _L), jnp.float32),
        pltpu.VMEM((_L, _L), jnp.float32),
        pltpu.SemaphoreType.DMA,
        pltpu.SemaphoreType.DMA,
        pltpu.SemaphoreType.DMA,
        pltpu.SemaphoreType.DMA,
        pltpu.SemaphoreType.DMA,
    ],
)


def _finish_body(n_ref, d_ref, o_ref):
    sn = jnp.sum(n_ref[...], axis=1, keepdims=True)
    dl = d_ref[...]
    l = jnp.log(jnp.exp(dl) + sn) - dl
    o_ref[0, 0] = jnp.sum(l) * (1.0 / _B)


def _finish(noise, dlab2):
    return pl.pallas_call(
        _finish_body,
        out_specs=pl.BlockSpec(memory_space=pltpu.SMEM),
        out_shape=jax.ShapeDtypeStruct((1, 1), jnp.float32),
    )(noise, dlab2)


def kernel(inputs, labels, projection):
    samples = _get_samples()
    projn = _prenorm_proj(projection).reshape(_VPAD, _D // 2)
    inpn = _prenorm(inputs, _B)
    noise, dlab = _sc_main(projn, inpn, labels.astype(jnp.int32), samples)
    loss = _finish(noise, dlab.reshape(_B, 1))
    return loss[0, 0]


# speedup vs baseline: 1.0537x; 1.0537x over previous
import numpy as np

import jax
import jax.numpy as jnp
from jax import lax
from jax.experimental import pallas as pl
from jax.experimental.pallas import tpu as pltpu
from jax.experimental.pallas import tpu_sc as plsc

_VOCAB = 1000000
_D = 64
_B = 1024
_NSAMP = 4096
_NCORES = 2
_NSUB = 16
_L = 16
_NW = _NCORES * _NSUB
_BPW = _B // _NW
_CHUNK = 128
_NCHUNK = _NSAMP // _CHUNK
_NBUF = 4

def _rotl(x, r):
    return (x << np.uint32(r)) | (x >> np.uint32(32 - r))


def _threefry2x32(k0, k1, x0, x1):
    x0 = x0.astype(np.uint32).copy()
    x1 = x1.astype(np.uint32).copy()
    ks = [np.uint32(k0), np.uint32(k1),
          np.uint32(k0) ^ np.uint32(k1) ^ np.uint32(0x1BD11BDA)]
    rotations = [(13, 15, 26, 6), (17, 29, 16, 24)]
    x0 += ks[0]
    x1 += ks[1]
    for i in range(5):
        for r in rotations[i % 2]:
            x0 += x1
            x1 = _rotl(x1, r)
            x1 ^= x0
        x0 += ks[(i + 1) % 3]
        x1 += ks[(i + 2) % 3] + np.uint32(i + 1)
    return x0, x1


def _random_bits32(k0, k1, size):
    b0, b1 = _threefry2x32(k0, k1, np.zeros(size, np.uint32),
                           np.arange(size, dtype=np.uint32))
    return b0 ^ b1


_samples_cache = []

_PB = 16384
_QB = _PB // 4
_NPB = (_VOCAB + _PB - 1) // _PB
_VPAD = _NPB * _PB


def _perm_np(j):
    i = j // _PB
    p = j % _PB
    return 4 * (i * _QB + p % _QB) + p // _QB


def _perm_vec(j):
    i = j // _PB
    p = j - i * _PB
    return 4 * (i * _QB + p % _QB) + p // _QB


def _get_samples():
    if not _samples_cache:
        b0, b1 = _threefry2x32(0, 1234, np.zeros(2, np.uint32),
                               np.arange(2, dtype=np.uint32))
        size = _B * _NSAMP
        hi = _random_bits32(b0[0], b1[0], size)
        lo = _random_bits32(b0[1], b1[1], size)
        span = np.uint32(_VOCAB)
        mult = np.uint32((2 ** 16) % _VOCAB)
        mult = np.uint32((int(mult) * int(mult)) % (2 ** 32)) % span
        off = ((hi % span) * mult + (lo % span)) % span
        _samples_cache.append(_perm_np(off.astype(np.int64)).astype(np.int32))
    return jnp.asarray(_samples_cache[0])


def _prenorm_body(x_ref, o_ref):
    x = x_ref[...]
    s = jnp.sum(x * x, axis=1, keepdims=True)
    o_ref[...] = x * lax.rsqrt(jnp.maximum(s, 1e-12))


def _prenorm(x, block_rows):
    rows = x.shape[0]
    grid = rows // block_rows
    return pl.pallas_call(
        _prenorm_body,
        grid=(grid,),
        in_specs=[pl.BlockSpec((block_rows, _D), lambda i: (i, 0))],
        out_specs=pl.BlockSpec((block_rows, _D), lambda i: (i, 0)),
        out_shape=jax.ShapeDtypeStruct((rows, _D), jnp.float32),
    )(x)


def _prenorm_proj_body(xt_ref, o_ref):
    xt = xt_ref[...]
    s = jnp.sum(xt * xt, axis=0, keepdims=True)
    xn = xt * lax.rsqrt(jnp.maximum(s, 1e-12))
    lo = lax.bitcast_convert_type(xn[0:_D // 2, :].astype(jnp.bfloat16),
                                  jnp.uint16).astype(jnp.int32)
    hi = lax.bitcast_convert_type(xn[_D // 2:_D, :].astype(jnp.bfloat16),
                                  jnp.uint16).astype(jnp.int32)
    packed = (lo | (hi << 16)).T
    for q in range(4):
        o_ref[:, q * 32:(q + 1) * 32] = packed[q * _QB:(q + 1) * _QB, :]


def _prenorm_proj(x):
    return pl.pallas_call(
        _prenorm_proj_body,
        grid=(_NPB,),
        in_specs=[pl.BlockSpec((_D, _PB), lambda i: (0, i))],
        out_specs=pl.BlockSpec((_QB, 128), lambda i: (i, 0)),
        out_shape=jax.ShapeDtypeStruct((_VPAD // 4, 128), jnp.int32),
    )(x.T)


def _sc_body(projn, inpn, labels, samples, noise_out, dlab_out,
             uin, lab, labrows, idx, rows0, rows1, rows2, rows3,
             nstage, dstage, scr, scr2, sem_l, s0, s1, s2, s3):
    w = lax.axis_index("s") * _NCORES + lax.axis_index("c")
    b0 = w * _BPW
    scrs = (scr, scr2)
    bufs = (rows0, rows1, rows2, rows3)
    sems = (s0, s1, s2, s3)
    iota = lax.iota(jnp.int32, _L)

    pltpu.sync_copy(inpn.at[pl.ds(b0, _BPW), :], uin)
    pltpu.sync_copy(labels.at[pl.ds(b0, _BPW)], lab)
    for g in range(_BPW // _L):
        lv = lab[pl.ds(g * _L, _L)]
        lab[pl.ds(g * _L, _L)] = _perm_vec(lv)
    pltpu.async_copy(projn.at[lab], labrows, sem_l).wait()

    def transpose_sum(S):
        dvs = []
        for j in range(_L):
            colrot = (iota + j) & (_L - 1)
            dvs.append(plsc.load_gather(S, [iota, colrot]))
        while len(dvs) > 1:
            dvs = [dvs[t] + dvs[t + 1] for t in range(0, len(dvs), 2)]
        return dvs[0]

    def unpack4(ref, row):
        a = ref[row, pl.ds(0, _L)]
        b = ref[row, pl.ds(_L, _L)]
        return (lax.bitcast_convert_type(lax.shift_left(a, 16), jnp.float32),
                lax.bitcast_convert_type(lax.shift_left(b, 16), jnp.float32),
                lax.bitcast_convert_type(a, jnp.float32),
                lax.bitcast_convert_type(b, jnp.float32))

    for g in range(_BPW // _L):
        for r in range(_L):
            row = g * _L + r
            vs = unpack4(labrows, row)
            pr = None
            for kd in range(_D // _L):
                uvec = uin[row, pl.ds(kd * _L, _L)]
                t = vs[kd] * uvec
                pr = t if pr is None else pr + t
            scr[r, :] = pr
        dstage[g, :] = transpose_sum(scr)

    def b_body(i, _):
        pltpu.sync_copy(samples.at[pl.ds((b0 + i) * _NSAMP, _NSAMP)], idx)
        us = [uin[i, pl.ds(k * _L, _L)] for k in range(_D // _L)]
        ua = plsc.pack(us[0], us[2], format=plsc.PackFormat.INTERLEAVED)
        ub = plsc.pack(us[1], us[3], format=plsc.PackFormat.INTERLEAVED)
        labv = plsc.load_gather(lab, [jnp.full((_L,), i, jnp.int32)])

        for k in range(_NBUF):
            pltpu.make_async_copy(projn.at[idx.at[pl.ds(k * _CHUNK, _CHUNK)]],
                                  bufs[k], sems[k]).start()

        def c_body(c0, acc):
            for k in range(_NBUF):
                c = c0 + k
                pltpu.make_async_copy(projn.at[idx.at[pl.ds(c * _CHUNK,
                                                            _CHUNK)]],
                                      bufs[k], sems[k]).wait()

                def g_body(m, acc2):
                    for half in range(2):
                        g = m * 2 + half
                        r0 = g * _L
                        sg = scrs[half]
                        for r8 in range(0, _L, 16):
                            avs = [plsc.bitcast(
                                bufs[k][r0 + r8 + t, pl.ds(0, _L)],
                                jnp.bfloat16) for t in range(16)]
                            bvs = [plsc.bitcast(
                                bufs[k][r0 + r8 + t, pl.ds(_L, _L)],
                                jnp.bfloat16) for t in range(16)]
                            ps = [avs[t] * ua for t in range(16)]
                            qs = [bvs[t] * ub for t in range(16)]
                            ss = [ps[t] + qs[t] for t in range(16)]
                            eo = [plsc.unpack(
                                ss[t], format=plsc.PackFormat.INTERLEAVED)
                                for t in range(16)]
                            prs = [eo[t][0] + eo[t][1] for t in range(16)]
                            for t in range(16):
                                sg[r8 + t, :] = prs[t]
                        dot = transpose_sum(sg)
                        ids16 = idx[pl.ds(c * _CHUNK + r0, _L)]
                        e = jnp.where(ids16 == labv, 0.0, jnp.exp(dot))
                        acc2 = acc2 + e
                    return acc2

                acc = lax.fori_loop(0, _CHUNK // (2 * _L), g_body, acc)

                nxt = c + _NBUF

                @pl.when(nxt < _NCHUNK)
                def _():
                    pltpu.make_async_copy(
                        projn.at[idx.at[pl.ds(nxt * _CHUNK, _CHUNK)]],
                        bufs[k], sems[k]).start()
            return acc

        acc = lax.fori_loop(0, _NCHUNK // _NBUF, c_body,
                            jnp.zeros((_L,), jnp.float32))
        nstage[i, :] = acc
        return 0

    lax.fori_loop(0, _BPW, b_body, 0)

    pltpu.sync_copy(nstage, noise_out.at[pl.ds(b0, _BPW), :])
    ng = _BPW // _L
    pltpu.sync_copy(dstage, dlab_out.at[pl.ds(w * ng, ng), :])


_sc_main = pl.kernel(
    _sc_body,
    out_type=(
        jax.ShapeDtypeStruct((_B, _L), jnp.float32),
        jax.ShapeDtypeStruct((_B // _L, _L), jnp.float32),
    ),
    mesh=plsc.VectorSubcoreMesh(core_axis_name="c", subcore_axis_name="s"),
    compiler_params=pltpu.CompilerParams(needs_layout_passes=False,
                                         use_tc_tiling_on_sc=False,
                                         disable_bounds_checks=True),
    scratch_types=[
        pltpu.VMEM((_BPW, _D), jnp.float32),
        pltpu.VMEM((_BPW,), jnp.int32),
        pltpu.VMEM((_BPW, _D // 2), jnp.int32),
        pltpu.VMEM((_NSAMP,), jnp.int32),
        pltpu.VMEM((_CHUNK, _D // 2), jnp.int32),
        pltpu.VMEM((_CHUNK, _D // 2), jnp.int32),
        pltpu.VMEM((_CHUNK, _D // 2), jnp.int32),
        pltpu.VMEM((_CHUNK, _D // 2), jnp.int32),
        pltpu.VMEM((_BPW, _L), jnp.float32),
        pltpu.VMEM((_BPW // _L, _L), jnp.float32),
        pltpu.VMEM((_L, _L), jnp.float32),
        pltpu.VMEM((_L, _L), jnp.float32),
        pltpu.SemaphoreType.DMA,
        pltpu.SemaphoreType.DMA,
        pltpu.SemaphoreType.DMA,
        pltpu.SemaphoreType.DMA,
        pltpu.SemaphoreType.DMA,
    ],
)


def _finish_body(n_ref, d_ref, o_ref):
    sn = jnp.sum(n_ref[...], axis=1, keepdims=True)
    dl = d_ref[...]
    l = jnp.log(jnp.exp(dl) + sn) - dl
    o_ref[0, 0] = jnp.sum(l) * (1.0 / _B)


def _finish(noise, dlab2):
    return pl.pallas_call(
        _finish_body,
        out_specs=pl.BlockSpec(memory_space=pltpu.SMEM),
        out_shape=jax.ShapeDtypeStruct((1, 1), jnp.float32),
    )(noise, dlab2)


def kernel(inputs, labels, projection):
    samples = _get_samples()
    projn = _prenorm_proj(projection).reshape(_VPAD, _D // 2)
    inpn = _prenorm(inputs, _B)
    noise, dlab = _sc_main(projn, inpn, labels.astype(jnp.int32), samples)
    loss = _finish(noise, dlab.reshape(_B, 1))
    return loss[0, 0]


# speedup vs baseline: 1.0883x; 1.0328x over previous
import numpy as np

import jax
import jax.numpy as jnp
from jax import lax
from jax.experimental import pallas as pl
from jax.experimental.pallas import tpu as pltpu
from jax.experimental.pallas import tpu_sc as plsc

_VOCAB = 1000000
_D = 64
_B = 1024
_NSAMP = 4096
_NCORES = 2
_NSUB = 16
_L = 16
_NW = _NCORES * _NSUB
_BPW = _B // _NW
_CHUNK = 128
_NCHUNK = _NSAMP // _CHUNK
_NBUF = 4

def _rotl(x, r):
    return (x << np.uint32(r)) | (x >> np.uint32(32 - r))


def _threefry2x32(k0, k1, x0, x1):
    x0 = x0.astype(np.uint32).copy()
    x1 = x1.astype(np.uint32).copy()
    ks = [np.uint32(k0), np.uint32(k1),
          np.uint32(k0) ^ np.uint32(k1) ^ np.uint32(0x1BD11BDA)]
    rotations = [(13, 15, 26, 6), (17, 29, 16, 24)]
    x0 += ks[0]
    x1 += ks[1]
    for i in range(5):
        for r in rotations[i % 2]:
            x0 += x1
            x1 = _rotl(x1, r)
            x1 ^= x0
        x0 += ks[(i + 1) % 3]
        x1 += ks[(i + 2) % 3] + np.uint32(i + 1)
    return x0, x1


def _random_bits32(k0, k1, size):
    b0, b1 = _threefry2x32(k0, k1, np.zeros(size, np.uint32),
                           np.arange(size, dtype=np.uint32))
    return b0 ^ b1


_samples_cache = []

_PB = 16384
_QB = _PB // 4
_NPB = (_VOCAB + _PB - 1) // _PB
_VPAD = _NPB * _PB


def _perm_np(j):
    i = j // _PB
    p = j % _PB
    return 4 * (i * _QB + p % _QB) + p // _QB


def _perm_vec(j):
    i = j // _PB
    p = j - i * _PB
    return 4 * (i * _QB + p % _QB) + p // _QB


def _get_samples():
    if not _samples_cache:
        b0, b1 = _threefry2x32(0, 1234, np.zeros(2, np.uint32),
                               np.arange(2, dtype=np.uint32))
        size = _B * _NSAMP
        hi = _random_bits32(b0[0], b1[0], size)
        lo = _random_bits32(b0[1], b1[1], size)
        span = np.uint32(_VOCAB)
        mult = np.uint32((2 ** 16) % _VOCAB)
        mult = np.uint32((int(mult) * int(mult)) % (2 ** 32)) % span
        off = ((hi % span) * mult + (lo % span)) % span
        _samples_cache.append(_perm_np(off.astype(np.int64)).astype(np.int32))
    return jnp.asarray(_samples_cache[0])


def _prenorm_body(x_ref, o_ref):
    x = x_ref[...]
    s = jnp.sum(x * x, axis=1, keepdims=True)
    o_ref[...] = x * lax.rsqrt(jnp.maximum(s, 1e-12))


def _prenorm(x, block_rows):
    rows = x.shape[0]
    grid = rows // block_rows
    return pl.pallas_call(
        _prenorm_body,
        grid=(grid,),
        in_specs=[pl.BlockSpec((block_rows, _D), lambda i: (i, 0))],
        out_specs=pl.BlockSpec((block_rows, _D), lambda i: (i, 0)),
        out_shape=jax.ShapeDtypeStruct((rows, _D), jnp.float32),
    )(x)


def _prenorm_proj_body(xt_ref, o_ref):
    xt = xt_ref[...]
    s = jnp.sum(xt * xt, axis=0, keepdims=True)
    xn = xt * lax.rsqrt(jnp.maximum(s, 1e-12))
    lo = lax.bitcast_convert_type(xn[0:_D // 2, :].astype(jnp.bfloat16),
                                  jnp.uint16).astype(jnp.int32)
    hi = lax.bitcast_convert_type(xn[_D // 2:_D, :].astype(jnp.bfloat16),
                                  jnp.uint16).astype(jnp.int32)
    packed = (lo | (hi << 16)).T
    for q in range(4):
        o_ref[:, q * 32:(q + 1) * 32] = packed[q * _QB:(q + 1) * _QB, :]


def _prenorm_proj(x):
    return pl.pallas_call(
        _prenorm_proj_body,
        grid=(_NPB,),
        in_specs=[pl.BlockSpec((_D, _PB), lambda i: (0, i))],
        out_specs=pl.BlockSpec((_QB, 128), lambda i: (i, 0)),
        out_shape=jax.ShapeDtypeStruct((_VPAD // 4, 128), jnp.int32),
    )(x.T)


def _sc_body(projn, inpn, labels, samples, noise_out, dlab_out,
             uin, lab, labrows, idx, idxb, rows0, rows1, rows2, rows3,
             nstage, dstage, scr, scr2, sem_l, s0, s1, s2, s3, si0, si1):
    w = lax.axis_index("s") * _NCORES + lax.axis_index("c")
    b0 = w * _BPW
    scrs = (scr, scr2)
    bufs = (rows0, rows1, rows2, rows3)
    sems = (s0, s1, s2, s3)
    iota = lax.iota(jnp.int32, _L)

    pltpu.sync_copy(inpn.at[pl.ds(b0, _BPW), :], uin)
    pltpu.sync_copy(labels.at[pl.ds(b0, _BPW)], lab)
    for g in range(_BPW // _L):
        lv = lab[pl.ds(g * _L, _L)]
        lab[pl.ds(g * _L, _L)] = _perm_vec(lv)
    pltpu.async_copy(projn.at[lab], labrows, sem_l).wait()

    def transpose_sum(S):
        dvs = []
        for j in range(_L):
            colrot = (iota + j) & (_L - 1)
            dvs.append(plsc.load_gather(S, [iota, colrot]))
        while len(dvs) > 1:
            dvs = [dvs[t] + dvs[t + 1] for t in range(0, len(dvs), 2)]
        return dvs[0]

    def unpack4(ref, row):
        a = ref[row, pl.ds(0, _L)]
        b = ref[row, pl.ds(_L, _L)]
        return (lax.bitcast_convert_type(lax.shift_left(a, 16), jnp.float32),
                lax.bitcast_convert_type(lax.shift_left(b, 16), jnp.float32),
                lax.bitcast_convert_type(a, jnp.float32),
                lax.bitcast_convert_type(b, jnp.float32))

    for g in range(_BPW // _L):
        for r in range(_L):
            row = g * _L + r
            vs = unpack4(labrows, row)
            pr = None
            for kd in range(_D // _L):
                uvec = uin[row, pl.ds(kd * _L, _L)]
                t = vs[kd] * uvec
                pr = t if pr is None else pr + t
            scr[r, :] = pr
        dstage[g, :] = transpose_sum(scr)

    def idx_copy(bi, ref, sem):
        return pltpu.make_async_copy(
            samples.at[pl.ds((b0 + bi) * _NSAMP, _NSAMP)], ref, sem)

    def process_b(i, idx):
        us = [uin[i, pl.ds(k * _L, _L)] for k in range(_D // _L)]
        ua = plsc.pack(us[0], us[2], format=plsc.PackFormat.INTERLEAVED)
        ub = plsc.pack(us[1], us[3], format=plsc.PackFormat.INTERLEAVED)
        labv = plsc.load_gather(lab, [jnp.full((_L,), i, jnp.int32)])

        for k in range(_NBUF):
            pltpu.make_async_copy(projn.at[idx.at[pl.ds(k * _CHUNK, _CHUNK)]],
                                  bufs[k], sems[k]).start()

        def c_body(c0, acc):
            for k in range(_NBUF):
                c = c0 + k
                pltpu.make_async_copy(projn.at[idx.at[pl.ds(c * _CHUNK,
                                                            _CHUNK)]],
                                      bufs[k], sems[k]).wait()

                def g_body(m, acc2):
                    for half in range(2):
                        g = m * 2 + half
                        r0 = g * _L
                        sg = scrs[half]
                        for r8 in range(0, _L, 16):
                            avs = [plsc.bitcast(
                                bufs[k][r0 + r8 + t, pl.ds(0, _L)],
                                jnp.bfloat16) for t in range(16)]
                            bvs = [plsc.bitcast(
                                bufs[k][r0 + r8 + t, pl.ds(_L, _L)],
                                jnp.bfloat16) for t in range(16)]
                            ps = [avs[t] * ua for t in range(16)]
                            qs = [bvs[t] * ub for t in range(16)]
                            ss = [ps[t] + qs[t] for t in range(16)]
                            eo = [plsc.unpack(
                                ss[t], format=plsc.PackFormat.INTERLEAVED)
                                for t in range(16)]
                            prs = [eo[t][0] + eo[t][1] for t in range(16)]
                            for t in range(16):
                                sg[r8 + t, :] = prs[t]
                        dot = transpose_sum(sg)
                        ids16 = idx[pl.ds(c * _CHUNK + r0, _L)]
                        e = jnp.where(ids16 == labv, 0.0, jnp.exp(dot))
                        acc2 = acc2 + e
                    return acc2

                acc = lax.fori_loop(0, _CHUNK // (2 * _L), g_body, acc)

                nxt = c + _NBUF

                @pl.when(nxt < _NCHUNK)
                def _():
                    pltpu.make_async_copy(
                        projn.at[idx.at[pl.ds(nxt * _CHUNK, _CHUNK)]],
                        bufs[k], sems[k]).start()
            return acc

        acc = lax.fori_loop(0, _NCHUNK // _NBUF, c_body,
                            jnp.zeros((_L,), jnp.float32))
        nstage[i, :] = acc

    idxs = (idx, idxb)
    isems = (si0, si1)
    idx_copy(0, idx, si0).start()

    def b2_body(m, _):
        for half in range(2):
            i = m * 2 + half
            idx_copy(i, idxs[half], isems[half]).wait()
            nb = i + 1

            @pl.when(nb < _BPW)
            def _():
                idx_copy(nb, idxs[1 - half], isems[1 - half]).start()

            process_b(i, idxs[half])
        return 0

    lax.fori_loop(0, _BPW // 2, b2_body, 0)

    pltpu.sync_copy(nstage, noise_out.at[pl.ds(b0, _BPW), :])
    ng = _BPW // _L
    pltpu.sync_copy(dstage, dlab_out.at[pl.ds(w * ng, ng), :])


_sc_main = pl.kernel(
    _sc_body,
    out_type=(
        jax.ShapeDtypeStruct((_B, _L), jnp.float32),
        jax.ShapeDtypeStruct((_B // _L, _L), jnp.float32),
    ),
    mesh=plsc.VectorSubcoreMesh(core_axis_name="c", subcore_axis_name="s"),
    compiler_params=pltpu.CompilerParams(needs_layout_passes=False,
                                         use_tc_tiling_on_sc=False,
                                         disable_bounds_checks=True),
    scratch_types=[
        pltpu.VMEM((_BPW, _D), jnp.float32),
        pltpu.VMEM((_BPW,), jnp.int32),
        pltpu.VMEM((_BPW, _D // 2), jnp.int32),
        pltpu.VMEM((_NSAMP,), jnp.int32),
        pltpu.VMEM((_NSAMP,), jnp.int32),
        pltpu.VMEM((_CHUNK, _D // 2), jnp.int32),
        pltpu.VMEM((_CHUNK, _D // 2), jnp.int32),
        pltpu.VMEM((_CHUNK, _D // 2), jnp.int32),
        pltpu.VMEM((_CHUNK, _D // 2), jnp.int32),
        pltpu.VMEM((_BPW, _L), jnp.float32),
        pltpu.VMEM((_BPW // _L, _L), jnp.float32),
        pltpu.VMEM((_L, _L), jnp.float32),
        pltpu.VMEM((_L, _L), jnp.float32),
        pltpu.SemaphoreType.DMA,
        pltpu.SemaphoreType.DMA,
        pltpu.SemaphoreType.DMA,
        pltpu.SemaphoreType.DMA,
        pltpu.SemaphoreType.DMA,
        pltpu.SemaphoreType.DMA,
        pltpu.SemaphoreType.DMA,
    ],
)


def _finish_body(n_ref, d_ref, o_ref):
    sn = jnp.sum(n_ref[...], axis=1, keepdims=True)
    dl = d_ref[...]
    l = jnp.log(jnp.exp(dl) + sn) - dl
    o_ref[0, 0] = jnp.sum(l) * (1.0 / _B)


def _finish(noise, dlab2):
    return pl.pallas_call(
        _finish_body,
        out_specs=pl.BlockSpec(memory_space=pltpu.SMEM),
        out_shape=jax.ShapeDtypeStruct((1, 1), jnp.float32),
    )(noise, dlab2)


def kernel(inputs, labels, projection):
    samples = _get_samples()
    projn = _prenorm_proj(projection).reshape(_VPAD, _D // 2)
    inpn = _prenorm(inputs, _B)
    noise, dlab = _sc_main(projn, inpn, labels.astype(jnp.int32), samples)
    loss = _finish(noise, dlab.reshape(_B, 1))
    return loss[0, 0]
